# baseline (speedup 1.0000x reference)
import functools
import jax
import jax.numpy as jnp
from jax import lax
from jax.experimental import pallas as pl
from jax.experimental.pallas import tpu as pltpu
from jax.experimental.pallas import tpu_sc as plsc

_DIM = 128
_NW = 32
_CHUNK = 128
_NSLOTS = 4


def _sc_kernel(rows):
    rows_per_w = rows // _NW
    n_chunks = rows_per_w // _CHUNK
    assert n_chunks % _NSLOTS == 0
    cwords = _CHUNK * _DIM
    mesh = plsc.VectorSubcoreMesh(core_axis_name="c", subcore_axis_name="s")

    @functools.partial(
        pl.kernel, mesh=mesh,
        out_type=jax.ShapeDtypeStruct((rows * _DIM,), jnp.float32),
        scratch_types=(
            [pltpu.VMEM((cwords,), jnp.float32)] * _NSLOTS
            + [pltpu.SemaphoreType.DMA] * (2 * _NSLOTS)
        ),
        compiler_params=pltpu.CompilerParams(needs_layout_passes=False),
    )
    def k(x_hbm, out_hbm, *scratch):
        bufs = scratch[:_NSLOTS]
        in_sems = scratch[_NSLOTS:2 * _NSLOTS]
        out_sems = scratch[2 * _NSLOTS:]
        wid = lax.axis_index("s") * 2 + lax.axis_index("c")
        base_w = wid * rows_per_w * _DIM
        iota = lax.iota(jnp.int32, 16)
        two = jnp.full((16,), 2.0, jnp.float32)

        def hslice(ref, ci):
            return ref.at[pl.ds(base_w + ci * cwords, cwords)]

        diag = [(iota + kk) & 15 for kk in range(16)]
        s16 = jnp.full((16,), 16, jnp.int32)

        def compute(buf):
            def tile_body(t, carry):
                word0 = (iota + t * 16) * _DIM

                def argmax16(b0):
                    wb = word0 + b0
                    vs = [plsc.load_gather(buf, [wb + diag[kk]])
                          for kk in range(16)]
                    mx = vs
                    while len(mx) > 1:
                        mx = [jnp.maximum(mx[i], mx[i + 1])
                              for i in range(0, len(mx), 2)]
                    cand = [jnp.where(vs[kk] == mx[0], diag[kk], s16)
                            for kk in range(16)]
                    while len(cand) > 1:
                        cand = [jnp.minimum(cand[i], cand[i + 1])
                                for i in range(0, len(cand), 2)]
                    return cand[0]

                m0 = plsc.load_gather(buf, [word0])
                m1 = plsc.load_gather(buf, [word0 + 1])
                ok = (m0 >= 0.5) & (m1 >= 0.5)

                byte_a = argmax16(16) + (argmax16(32) << 4)
                byte_b = argmax16(48) + (argmax16(64) << 4)
                prod = (byte_a * byte_b) & 255
                plsc.addupdate_scatter(buf, [word0 + 80 + (prod & 15)],
                                       two, mask=ok)
                plsc.addupdate_scatter(buf, [word0 + 96 + (prod >> 4)],
                                       two, mask=ok)
                return carry

            lax.fori_loop(0, _CHUNK // 16, tile_body, 0)

        pltpu.async_copy(hslice(x_hbm, 0), bufs[0], in_sems[0])
        pltpu.async_copy(hslice(x_hbm, 1), bufs[1], in_sems[1])

        def round_body(kk, carry):
            for off in range(_NSLOTS):
                ci = kk * _NSLOTS + off
                s = off
                s2 = (off + 2) % _NSLOTS
                pltpu.make_async_copy(hslice(x_hbm, ci), bufs[s],
                                      in_sems[s]).wait()

                @pl.when(ci >= 2)
                def _():
                    pltpu.make_async_copy(bufs[s2], hslice(out_hbm, ci - 2),
                                          out_sems[s2]).wait()

                @pl.when(ci + 2 < n_chunks)
                def _():
                    pltpu.async_copy(hslice(x_hbm, ci + 2), bufs[s2],
                                     in_sems[s2])

                compute(bufs[s])
                pltpu.async_copy(bufs[s], hslice(out_hbm, ci), out_sems[s])
            return carry

        lax.fori_loop(0, n_chunks // _NSLOTS, round_body, 0)

        for ci in (n_chunks - 2, n_chunks - 1):
            s = ci % _NSLOTS
            pltpu.make_async_copy(bufs[s], hslice(out_hbm, ci),
                                  out_sems[s]).wait()

    return k


def kernel(x_bd):
    b, s, d = x_bd.shape
    rows = b * s
    x2 = x_bd.reshape(rows * d)
    out = _sc_kernel(rows)(x2)
    return out.reshape(b, s, d)
